# Initial kernel scaffold; baseline (speedup 1.0000x reference)
import functools

import jax
import jax.numpy as jnp
from jax import lax
from jax.experimental import pallas as pl
from jax.experimental.pallas import tpu as pltpu
from jax.experimental.pallas import tpu_sc as plsc

N = 10000
E = 160000
NUM_NUMERIC = 128
EMBED = 128
HIDDEN = 256
NCAT = 8
CARD = 1001

NPAD = 10240
HEXT = HIDDEN + 16
DEG_COL = HIDDEN

NC = 2
NS = 16
LANES = 16

NODES_PER_TILE = NPAD // (NC * NS)
G_CHUNK = 128
CAT_ROWS_PER_TILE = NODES_PER_TILE * NCAT
CAT_CHUNKS = CAT_ROWS_PER_TILE // G_CHUNK

HALF = NPAD // NC
ACC_ROWS = HALF + 8
TRASH = HALF

E_CHUNKS = -(-E // (NC * NS * G_CHUNK))
E_PER_TILE = E_CHUNKS * G_CHUNK
E_PAD = E_PER_TILE * NS

ROW_BLK = 512
N_BLKS = NPAD // ROW_BLK


def _sc_gather_body(xcatf, tabf, out, idxraw, idxg0, idxg1, row0, row1,
                    sem0, sem1):
    c = lax.axis_index("c")
    s = lax.axis_index("s")
    wid = s * NC + c
    base = wid * CAT_ROWS_PER_TILE
    col_off = (lax.broadcasted_iota(jnp.int32, (LANES,), 0) % NCAT) * CARD

    bufs = [(idxg0, row0, sem0), (idxg1, row1, sem1)]
    descs = [None, None]
    for j in range(CAT_CHUNKS + 1):
        if j < CAT_CHUNKS:
            idxg, rowb, sem = bufs[j % 2]
            off = base + j * G_CHUNK
            pltpu.sync_copy(xcatf.at[pl.ds(off, G_CHUNK)], idxraw)
            for i in range(G_CHUNK // LANES):
                v = idxraw[pl.ds(i * LANES, LANES)]
                v = jnp.minimum(jnp.maximum(v, 0), CARD - 1) + col_off
                idxg[pl.ds(i * LANES, LANES)] = v
            descs[j % 2] = pltpu.async_copy(tabf.at[idxg], rowb, sem)
        if j >= 1:
            k = j - 1
            idxg, rowb, sem = bufs[k % 2]
            descs[k % 2].wait()
            pltpu.sync_copy(rowb, out.at[pl.ds(base + k * G_CHUNK, G_CHUNK)])


_sc_gather = functools.partial(
    pl.kernel,
    out_type=jax.ShapeDtypeStruct((NPAD * NCAT, EMBED), jnp.float32),
    mesh=plsc.VectorSubcoreMesh(core_axis_name="c", subcore_axis_name="s"),
    scratch_types=[
        pltpu.VMEM((G_CHUNK,), jnp.int32),
        pltpu.VMEM((G_CHUNK,), jnp.int32),
        pltpu.VMEM((G_CHUNK,), jnp.int32),
        pltpu.VMEM((G_CHUNK, EMBED), jnp.float32),
        pltpu.VMEM((G_CHUNK, EMBED), jnp.float32),
        pltpu.SemaphoreType.DMA,
        pltpu.SemaphoreType.DMA,
    ],
)(_sc_gather_body)


def _sc_agg_body(edges, hext, zrows, agg, srcall, dstall, dloc,
                 pay0, pay1, sem0, sem1, acc):
    c = lax.axis_index("c")
    s = lax.axis_index("s")
    cbase = c * HALF
    ebase = s * E_PER_TILE

    pltpu.sync_copy(zrows, acc.at[pl.ds(s * NODES_PER_TILE, NODES_PER_TILE)])

    @pl.when(s == NS - 1)
    def _():
        pltpu.sync_copy(zrows.at[pl.ds(0, ACC_ROWS - HALF)],
                        acc.at[pl.ds(HALF, ACC_ROWS - HALF)])

    pltpu.sync_copy(edges.at[0, pl.ds(ebase, E_PER_TILE)], srcall)
    pltpu.sync_copy(edges.at[1, pl.ds(ebase, E_PER_TILE)], dstall)

    plsc.subcore_barrier()

    def rowbody(r, carry):
        for i in range(G_CHUNK // LANES):
            off = pl.multiple_of(r * G_CHUNK + i * LANES, LANES)
            v = dstall[pl.ds(off, LANES)]
            lv = v - cbase
            ok = (lv >= 0) & (lv < HALF)
            lv = jnp.where(ok, lv, TRASH + (v & 7))
            dloc[r, pl.ds(i * LANES, LANES)] = lv
        return carry

    lax.fori_loop(0, E_CHUNKS, rowbody, 0)

    bufs = [(pay0, sem0), (pay1, sem1)]
    descs = [None, None]
    for j in range(E_CHUNKS + 1):
        if j < E_CHUNKS:
            pay, sem = bufs[j % 2]
            idx = srcall.at[pl.ds(j * G_CHUNK, G_CHUNK)]
            descs[j % 2] = pltpu.async_copy(hext.at[idx], pay, sem)
        if j >= 1:
            k = j - 1
            pay, sem = bufs[k % 2]
            descs[k % 2].wait()
            pltpu.sync_copy(pay, acc.at[dloc.at[k]], add=True)

    plsc.subcore_barrier()
    pltpu.sync_copy(
        acc.at[pl.ds(s * NODES_PER_TILE, NODES_PER_TILE)],
        agg.at[pl.ds(cbase + s * NODES_PER_TILE, NODES_PER_TILE)])


_sc_agg = functools.partial(
    pl.kernel,
    out_type=jax.ShapeDtypeStruct((NPAD, HEXT), jnp.float32),
    mesh=plsc.VectorSubcoreMesh(core_axis_name="c", subcore_axis_name="s"),
    scratch_types=[
        pltpu.VMEM((E_PER_TILE,), jnp.int32),
        pltpu.VMEM((E_PER_TILE,), jnp.int32),
        pltpu.VMEM((E_CHUNKS, G_CHUNK), jnp.int32),
        pltpu.VMEM((G_CHUNK, HEXT), jnp.float32),
        pltpu.VMEM((G_CHUNK, HEXT), jnp.float32),
        pltpu.SemaphoreType.DMA,
        pltpu.SemaphoreType.DMA,
        pltpu.VMEM_SHARED((ACC_ROWS, HEXT), jnp.float32),
    ],
)(_sc_agg_body)


def _tc_in_body(xn_ref, cat_ref, wn_ref, bn_ref, wi_ref, bi_ref, out_ref):
    t = jnp.dot(xn_ref[...], wn_ref[...], preferred_element_type=jnp.float32)
    t = jnp.maximum(t + bn_ref[...][None, :], 0.0)
    acc = jnp.dot(t, wi_ref[0:EMBED, :], preferred_element_type=jnp.float32)
    acc += jnp.dot(cat_ref[...], wi_ref[EMBED:, :],
                   preferred_element_type=jnp.float32)
    h = jnp.maximum(acc + bi_ref[...][None, :], 0.0)
    out_ref[:, 0:HIDDEN] = h
    tail = lax.broadcasted_iota(jnp.int32, (ROW_BLK, HEXT - HIDDEN), 1)
    out_ref[:, HIDDEN:] = jnp.where(tail == 0, 1.0, 0.0)


def _tc_in(x_num, cat_concat, W_num, b_num, W_in, b_in):
    return pl.pallas_call(
        _tc_in_body,
        grid=(N_BLKS,),
        in_specs=[
            pl.BlockSpec((ROW_BLK, NUM_NUMERIC), lambda i: (i, 0)),
            pl.BlockSpec((ROW_BLK, NCAT * EMBED), lambda i: (i, 0)),
            pl.BlockSpec((NUM_NUMERIC, EMBED), lambda i: (0, 0)),
            pl.BlockSpec((EMBED,), lambda i: (0,)),
            pl.BlockSpec((EMBED * (1 + NCAT), HIDDEN), lambda i: (0, 0)),
            pl.BlockSpec((HIDDEN,), lambda i: (0,)),
        ],
        out_specs=pl.BlockSpec((ROW_BLK, HEXT), lambda i: (i, 0)),
        out_shape=jax.ShapeDtypeStruct((NPAD, HEXT), jnp.float32),
    )(x_num, cat_concat, W_num, b_num, W_in, b_in)


def _tc_out_body(agg_ref, wg_ref, bg_ref, wo_ref, bo_ref, out_ref):
    a = agg_ref[:, 0:HIDDEN]
    deg = jnp.maximum(agg_ref[:, DEG_COL:DEG_COL + 1], 1.0)
    t = jnp.dot(a, wg_ref[...], preferred_element_type=jnp.float32)
    h = jnp.maximum(t / deg + bg_ref[...][None, :], 0.0)
    o = jnp.sum(h * wo_ref[...][None, :], axis=1) + bo_ref[0]
    out_ref[...] = o


def _tc_out(agg, W_gcn, b_gcn, w_out_vec, b_out):
    return pl.pallas_call(
        _tc_out_body,
        grid=(N_BLKS,),
        in_specs=[
            pl.BlockSpec((ROW_BLK, HEXT), lambda i: (i, 0)),
            pl.BlockSpec((HIDDEN, HIDDEN), lambda i: (0, 0)),
            pl.BlockSpec((HIDDEN,), lambda i: (0,)),
            pl.BlockSpec((HIDDEN,), lambda i: (0,)),
            pl.BlockSpec((1,), lambda i: (0,)),
        ],
        out_specs=pl.BlockSpec((ROW_BLK,), lambda i: (i,)),
        out_shape=jax.ShapeDtypeStruct((NPAD,), jnp.float32),
    )(agg, W_gcn, b_gcn, w_out_vec, b_out)


def kernel(x_num, x_cat, edge_index, tables, W_num, b_num, W_in, b_in,
           W_gcn, b_gcn, W_out, b_out):
    x_num_p = jnp.zeros((NPAD, NUM_NUMERIC), jnp.float32).at[:N].set(x_num)
    x_cat_p = jnp.zeros((NPAD, NCAT), jnp.int32).at[:N].set(
        x_cat.astype(jnp.int32))
    xcat_flat = x_cat_p.reshape(NPAD * NCAT)
    tab_flat = tables.reshape(NCAT * CARD, EMBED)

    src = edge_index[0].astype(jnp.int32)
    dst = edge_index[1].astype(jnp.int32)
    src_p = jnp.zeros((E_PAD,), jnp.int32).at[:E].set(src)
    dst_p = jnp.full((E_PAD,), -8, jnp.int32).at[:E].set(dst)
    edges = jnp.stack([src_p, dst_p])

    zrows = jnp.zeros((NODES_PER_TILE, HEXT), jnp.float32)

    cat_flat = _sc_gather(xcat_flat, tab_flat)
    cat_concat = cat_flat.reshape(NPAD, NCAT * EMBED)

    hext = _tc_in(x_num_p, cat_concat, W_num, b_num, W_in, b_in)

    agg = _sc_agg(edges, hext, zrows)

    out = _tc_out(agg, W_gcn, b_gcn, W_out[:, 0], b_out)
    return out[:N]


# speedup vs baseline: 2.6596x; 2.6596x over previous
import functools

import jax
import jax.numpy as jnp
from jax import lax
from jax.experimental import pallas as pl
from jax.experimental.pallas import tpu as pltpu
from jax.experimental.pallas import tpu_sc as plsc

N = 10000
E = 160000
NUM_NUMERIC = 128
EMBED = 128
HIDDEN = 256
NCAT = 8
CARD = 1001

NPAD = 10240
HEXT = HIDDEN + 16
DEG_COL = HIDDEN

NC = 2
NS = 16
LANES = 16

NODES_PER_TILE = NPAD // (NC * NS)
G_CHUNK = 128
CAT_ROWS_PER_TILE = NODES_PER_TILE * NCAT
CAT_CHUNKS = CAT_ROWS_PER_TILE // G_CHUNK

HALF = NPAD // NC
ACC_ROWS = HALF + 8
TRASH = HALF

E_CHUNK = 64
E_CHUNKS = 2 * (-(-E // (NS * E_CHUNK * 2)))
E_PER_TILE = E_CHUNKS * E_CHUNK
E_PAD = E_PER_TILE * NS

ROW_BLK = 512
N_BLKS = NPAD // ROW_BLK


def _sc_gather_body(xcatf, tabf, out, idxraw, idxg0, idxg1, row0, row1,
                    sem0, sem1):
    c = lax.axis_index("c")
    s = lax.axis_index("s")
    wid = s * NC + c
    base = wid * CAT_ROWS_PER_TILE
    col_off = (lax.broadcasted_iota(jnp.int32, (LANES,), 0) % NCAT) * CARD

    bufs = [(idxg0, row0, sem0), (idxg1, row1, sem1)]
    descs = [None, None]
    for j in range(CAT_CHUNKS + 1):
        if j < CAT_CHUNKS:
            idxg, rowb, sem = bufs[j % 2]
            off = base + j * G_CHUNK
            pltpu.sync_copy(xcatf.at[pl.ds(off, G_CHUNK)], idxraw)
            for i in range(G_CHUNK // LANES):
                v = idxraw[pl.ds(i * LANES, LANES)]
                v = jnp.minimum(jnp.maximum(v, 0), CARD - 1) + col_off
                idxg[pl.ds(i * LANES, LANES)] = v
            descs[j % 2] = pltpu.async_copy(tabf.at[idxg], rowb, sem)
        if j >= 1:
            k = j - 1
            idxg, rowb, sem = bufs[k % 2]
            descs[k % 2].wait()
            pltpu.sync_copy(rowb, out.at[pl.ds(base + k * G_CHUNK, G_CHUNK)])


_sc_gather = functools.partial(
    pl.kernel,
    out_type=jax.ShapeDtypeStruct((NPAD * NCAT, EMBED), jnp.float32),
    mesh=plsc.VectorSubcoreMesh(core_axis_name="c", subcore_axis_name="s"),
    scratch_types=[
        pltpu.VMEM((G_CHUNK,), jnp.int32),
        pltpu.VMEM((G_CHUNK,), jnp.int32),
        pltpu.VMEM((G_CHUNK,), jnp.int32),
        pltpu.VMEM((G_CHUNK, EMBED), jnp.float32),
        pltpu.VMEM((G_CHUNK, EMBED), jnp.float32),
        pltpu.SemaphoreType.DMA,
        pltpu.SemaphoreType.DMA,
    ],
)(_sc_gather_body)


def _localize(dstb, dlocb, cbase, n):
    for i in range(n // LANES):
        v = dstb[pl.ds(i * LANES, LANES)]
        lv = v - cbase
        ok = (lv >= 0) & (lv < HALF)
        lv = jnp.where(ok, lv, TRASH + (v & 7))
        dlocb[0, pl.ds(i * LANES, LANES)] = lv


def _sc_agg_body(edges, hext, agg, src0, src1, dst0, dst1, dloc0, dloc1,
                 pay0, pay1, sem0, sem1, zbuf, acc):
    c = lax.axis_index("c")
    s = lax.axis_index("s")
    cbase = c * HALF
    ebase = s * E_PER_TILE

    zv = jnp.zeros((LANES,), jnp.float32)
    for r in range(8):
        for i in range(HEXT // LANES):
            zbuf[r, pl.ds(i * LANES, LANES)] = zv
    for t in range(NODES_PER_TILE // 8):
        pltpu.sync_copy(zbuf, acc.at[pl.ds(s * NODES_PER_TILE + t * 8, 8)])

    @pl.when(s == NS - 1)
    def _():
        pltpu.sync_copy(zbuf, acc.at[pl.ds(HALF, ACC_ROWS - HALF)])

    plsc.subcore_barrier()

    bufs = [(src0, dst0, dloc0, pay0, sem0), (src1, dst1, dloc1, pay1, sem1)]

    def prologue(j, srcb, dstb, dlocb):
        eoff = pl.multiple_of(ebase + j * E_CHUNK, E_CHUNK)
        pltpu.sync_copy(edges.at[0, pl.ds(eoff, E_CHUNK)], srcb)
        pltpu.sync_copy(edges.at[1, pl.ds(eoff, E_CHUNK)], dstb)
        _localize(dstb, dlocb, cbase, E_CHUNK)

    def pairbody(p, carry):
        j0 = p * 2
        src0b, dst0b, dloc0b, pay0b, sem0b = bufs[0]
        src1b, dst1b, dloc1b, pay1b, sem1b = bufs[1]
        prologue(j0, src0b, dst0b, dloc0b)
        d0 = pltpu.async_copy(hext.at[src0b], pay0b, sem0b)
        prologue(j0 + 1, src1b, dst1b, dloc1b)
        d1 = pltpu.async_copy(hext.at[src1b], pay1b, sem1b)
        d0.wait()
        pltpu.sync_copy(pay0b, acc.at[dloc0b.at[0]], add=True)
        d1.wait()
        pltpu.sync_copy(pay1b, acc.at[dloc1b.at[0]], add=True)
        return carry

    lax.fori_loop(0, E_CHUNKS // 2, pairbody, 0)

    plsc.subcore_barrier()
    pltpu.sync_copy(
        acc.at[pl.ds(s * NODES_PER_TILE, NODES_PER_TILE)],
        agg.at[pl.ds(cbase + s * NODES_PER_TILE, NODES_PER_TILE)])


_sc_agg = functools.partial(
    pl.kernel,
    out_type=jax.ShapeDtypeStruct((NPAD, HEXT), jnp.float32),
    mesh=plsc.VectorSubcoreMesh(core_axis_name="c", subcore_axis_name="s"),
    scratch_types=[
        pltpu.VMEM((E_CHUNK,), jnp.int32),
        pltpu.VMEM((E_CHUNK,), jnp.int32),
        pltpu.VMEM((E_CHUNK,), jnp.int32),
        pltpu.VMEM((E_CHUNK,), jnp.int32),
        pltpu.VMEM((1, E_CHUNK), jnp.int32),
        pltpu.VMEM((1, E_CHUNK), jnp.int32),
        pltpu.VMEM((E_CHUNK, HEXT), jnp.float32),
        pltpu.VMEM((E_CHUNK, HEXT), jnp.float32),
        pltpu.SemaphoreType.DMA,
        pltpu.SemaphoreType.DMA,
        pltpu.VMEM((8, HEXT), jnp.float32),
        pltpu.VMEM_SHARED((ACC_ROWS, HEXT), jnp.float32),
    ],
    compiler_params=pltpu.CompilerParams(use_tc_tiling_on_sc=False),
)(_sc_agg_body)


def _tc_in_body(xn_ref, cat_ref, wn_ref, bn_ref, wi_ref, bi_ref, out_ref):
    t = jnp.dot(xn_ref[...], wn_ref[...], preferred_element_type=jnp.float32)
    t = jnp.maximum(t + bn_ref[...][None, :], 0.0)
    acc = jnp.dot(t, wi_ref[0:EMBED, :], preferred_element_type=jnp.float32)
    acc += jnp.dot(cat_ref[...], wi_ref[EMBED:, :],
                   preferred_element_type=jnp.float32)
    h = jnp.maximum(acc + bi_ref[...][None, :], 0.0)
    out_ref[:, 0:HIDDEN] = h
    tail = lax.broadcasted_iota(jnp.int32, (ROW_BLK, HEXT - HIDDEN), 1)
    out_ref[:, HIDDEN:] = jnp.where(tail == 0, 1.0, 0.0)


def _tc_in(x_num, cat_concat, W_num, b_num, W_in, b_in):
    return pl.pallas_call(
        _tc_in_body,
        grid=(N_BLKS,),
        in_specs=[
            pl.BlockSpec((ROW_BLK, NUM_NUMERIC), lambda i: (i, 0)),
            pl.BlockSpec((ROW_BLK, NCAT * EMBED), lambda i: (i, 0)),
            pl.BlockSpec((NUM_NUMERIC, EMBED), lambda i: (0, 0)),
            pl.BlockSpec((EMBED,), lambda i: (0,)),
            pl.BlockSpec((EMBED * (1 + NCAT), HIDDEN), lambda i: (0, 0)),
            pl.BlockSpec((HIDDEN,), lambda i: (0,)),
        ],
        out_specs=pl.BlockSpec((ROW_BLK, HEXT), lambda i: (i, 0)),
        out_shape=jax.ShapeDtypeStruct((NPAD, HEXT), jnp.float32),
    )(x_num, cat_concat, W_num, b_num, W_in, b_in)


def _tc_out_body(agg_ref, wg_ref, bg_ref, wo_ref, bo_ref, out_ref):
    a = agg_ref[:, 0:HIDDEN]
    deg = jnp.maximum(agg_ref[:, DEG_COL:DEG_COL + 1], 1.0)
    t = jnp.dot(a, wg_ref[...], preferred_element_type=jnp.float32)
    h = jnp.maximum(t / deg + bg_ref[...][None, :], 0.0)
    o = jnp.sum(h * wo_ref[...][None, :], axis=1) + bo_ref[0]
    out_ref[...] = o


def _tc_out(agg, W_gcn, b_gcn, w_out_vec, b_out):
    return pl.pallas_call(
        _tc_out_body,
        grid=(N_BLKS,),
        in_specs=[
            pl.BlockSpec((ROW_BLK, HEXT), lambda i: (i, 0)),
            pl.BlockSpec((HIDDEN, HIDDEN), lambda i: (0, 0)),
            pl.BlockSpec((HIDDEN,), lambda i: (0,)),
            pl.BlockSpec((HIDDEN,), lambda i: (0,)),
            pl.BlockSpec((128,), lambda i: (0,)),
        ],
        out_specs=pl.BlockSpec((ROW_BLK,), lambda i: (i,)),
        out_shape=jax.ShapeDtypeStruct((NPAD,), jnp.float32),
    )(agg, W_gcn, b_gcn, w_out_vec, b_out)


def kernel(x_num, x_cat, edge_index, tables, W_num, b_num, W_in, b_in,
           W_gcn, b_gcn, W_out, b_out):
    x_num_p = jnp.zeros((NPAD, NUM_NUMERIC), jnp.float32).at[:N].set(x_num)
    x_cat_p = jnp.zeros((NPAD, NCAT), jnp.int32).at[:N].set(
        x_cat.astype(jnp.int32))
    xcat_flat = x_cat_p.reshape(NPAD * NCAT)
    tab_flat = tables.reshape(NCAT * CARD, EMBED)

    src = edge_index[0].astype(jnp.int32)
    dst = edge_index[1].astype(jnp.int32)
    src_p = jnp.zeros((E_PAD,), jnp.int32).at[:E].set(src)
    dst_p = jnp.full((E_PAD,), -8, jnp.int32).at[:E].set(dst)
    edges = jnp.stack([src_p, dst_p])

    cat_flat = _sc_gather(xcat_flat, tab_flat)
    cat_concat = cat_flat.reshape(NPAD, NCAT * EMBED)

    hext = _tc_in(x_num_p, cat_concat, W_num, b_num, W_in, b_in)

    agg = _sc_agg(edges, hext)

    out = _tc_out(agg, W_gcn, b_gcn, W_out[:, 0],
                  jnp.broadcast_to(b_out, (128,)))
    return out[:N]


# speedup vs baseline: 3.9046x; 1.4681x over previous
import functools

import jax
import jax.numpy as jnp
from jax import lax
from jax.experimental import pallas as pl
from jax.experimental.pallas import tpu as pltpu
from jax.experimental.pallas import tpu_sc as plsc

N = 10000
E = 160000
NUM_NUMERIC = 128
EMBED = 128
HIDDEN = 256
NCAT = 8
CARD = 1001

NPAD = 10240
HEXT = HIDDEN + 16
DEG_COL = HIDDEN

NC = 2
NS = 16
LANES = 16

NODES_PER_TILE = NPAD // (NC * NS)
G_CHUNK = 128
CAT_ROWS_PER_TILE = NODES_PER_TILE * NCAT
CAT_CHUNKS = CAT_ROWS_PER_TILE // G_CHUNK

HALF = NPAD // NC
ACC_ROWS = HALF + 8
TRASH = HALF

E_CHUNK = 48
IDX_CHUNK = 1264
E_PER_TILE = -(-E // (NS * IDX_CHUNK)) * IDX_CHUNK
E_PAD = E_PER_TILE * NS
PACK_SHIFT = 14
PACK = 1 << PACK_SHIFT

ROW_BLK = 512
N_BLKS = NPAD // ROW_BLK


def _sc_gather_body(xcatf, tabf, out, idxraw, idxg0, idxg1, row0, row1,
                    sem0, sem1):
    c = lax.axis_index("c")
    s = lax.axis_index("s")
    wid = s * NC + c
    base = wid * CAT_ROWS_PER_TILE
    col_off = (lax.broadcasted_iota(jnp.int32, (LANES,), 0) % NCAT) * CARD

    bufs = [(idxg0, row0, sem0), (idxg1, row1, sem1)]
    descs = [None, None]
    for j in range(CAT_CHUNKS + 1):
        if j < CAT_CHUNKS:
            idxg, rowb, sem = bufs[j % 2]
            off = base + j * G_CHUNK
            pltpu.sync_copy(xcatf.at[pl.ds(off, G_CHUNK)], idxraw)
            for i in range(G_CHUNK // LANES):
                v = idxraw[pl.ds(i * LANES, LANES)]
                v = jnp.minimum(jnp.maximum(v, 0), CARD - 1) + col_off
                idxg[pl.ds(i * LANES, LANES)] = v
            descs[j % 2] = pltpu.async_copy(tabf.at[idxg], rowb, sem)
        if j >= 1:
            k = j - 1
            idxg, rowb, sem = bufs[k % 2]
            descs[k % 2].wait()
            pltpu.sync_copy(rowb, out.at[pl.ds(base + k * G_CHUNK, G_CHUNK)])


_sc_gather = functools.partial(
    pl.kernel,
    out_type=jax.ShapeDtypeStruct((NPAD * NCAT, EMBED), jnp.float32),
    mesh=plsc.VectorSubcoreMesh(core_axis_name="c", subcore_axis_name="s"),
    scratch_types=[
        pltpu.VMEM((G_CHUNK,), jnp.int32),
        pltpu.VMEM((G_CHUNK,), jnp.int32),
        pltpu.VMEM((G_CHUNK,), jnp.int32),
        pltpu.VMEM((G_CHUNK, EMBED), jnp.float32),
        pltpu.VMEM((G_CHUNK, EMBED), jnp.float32),
        pltpu.SemaphoreType.DMA,
        pltpu.SemaphoreType.DMA,
    ],
)(_sc_gather_body)


def _sc_agg_body(edges, hext, agg, srcc, dstc, packed, sidx0, sidx1,
                 dloc0, dloc1, pay0, pay1, sem0, sem1, zbuf, acc):
    c = lax.axis_index("c")
    s = lax.axis_index("s")
    cbase = c * HALF
    ebase = s * E_PER_TILE

    zv = jnp.zeros((LANES,), jnp.float32)
    for r in range(8):
        for i in range(HEXT // LANES):
            zbuf[r, pl.ds(i * LANES, LANES)] = zv
    for t in range(NODES_PER_TILE // 8):
        pltpu.sync_copy(zbuf, acc.at[pl.ds(s * NODES_PER_TILE + t * 8, 8)])

    @pl.when(s == NS - 1)
    def _():
        pltpu.sync_copy(zbuf, acc.at[pl.ds(HALF, ACC_ROWS - HALF)])

    def cblock(q, cursor):
        eoff = pl.multiple_of(ebase + q * IDX_CHUNK, LANES)
        pltpu.sync_copy(edges.at[0, pl.ds(eoff, IDX_CHUNK)], srcc)
        pltpu.sync_copy(edges.at[1, pl.ds(eoff, IDX_CHUNK)], dstc)

        def step(i, cur):
            srcv = srcc[pl.ds(i * LANES, LANES)]
            dstv = dstc[pl.ds(i * LANES, LANES)]
            lv = dstv - cbase
            ok = (lv >= 0) & (lv < HALF)
            pk = srcv * PACK + jnp.where(ok, lv, 0)
            oki = ok.astype(jnp.int32)
            pos = plsc.cumsum(oki)
            plsc.store_scatter(packed, [cur + pos - oki], pk, mask=ok)
            return cur + jnp.max(pos)

        return lax.fori_loop(0, IDX_CHUNK // LANES, step, cursor)

    cursor = lax.fori_loop(0, E_PER_TILE // IDX_CHUNK, cblock, 0)

    trash_pk = TRASH + (lax.broadcasted_iota(jnp.int32, (LANES,), 0) & 7)
    for k in range(2 * E_CHUNK // LANES):
        packed[pl.ds(cursor + k * LANES, LANES)] = trash_pk
    npairs = (cursor + 2 * E_CHUNK - 1) // (2 * E_CHUNK)

    plsc.subcore_barrier()

    bufs = [(sidx0, dloc0, pay0, sem0), (sidx1, dloc1, pay1, sem1)]

    def pairbody(p, carry):
        descs = [None, None]
        for b in range(2):
            sidx, dloc, pay, sem = bufs[b]
            off = pl.multiple_of(p * 2 * E_CHUNK + b * E_CHUNK, E_CHUNK)
            for i in range(E_CHUNK // LANES):
                pk = packed[pl.ds(off + i * LANES, LANES)]
                sidx[pl.ds(i * LANES, LANES)] = lax.shift_right_logical(
                    pk, PACK_SHIFT)
                dloc[0, pl.ds(i * LANES, LANES)] = pk & (PACK - 1)
            descs[b] = pltpu.async_copy(hext.at[sidx], pay, sem)
        for b in range(2):
            sidx, dloc, pay, sem = bufs[b]
            descs[b].wait()
            pltpu.sync_copy(pay, acc.at[dloc.at[0]], add=True)
        return carry

    lax.fori_loop(0, npairs, pairbody, 0)

    plsc.subcore_barrier()
    pltpu.sync_copy(
        acc.at[pl.ds(s * NODES_PER_TILE, NODES_PER_TILE)],
        agg.at[pl.ds(cbase + s * NODES_PER_TILE, NODES_PER_TILE)])


_sc_agg = functools.partial(
    pl.kernel,
    out_type=jax.ShapeDtypeStruct((NPAD, HEXT), jnp.float32),
    mesh=plsc.VectorSubcoreMesh(core_axis_name="c", subcore_axis_name="s"),
    scratch_types=[
        pltpu.VMEM((IDX_CHUNK,), jnp.int32),
        pltpu.VMEM((IDX_CHUNK,), jnp.int32),
        pltpu.VMEM((E_PER_TILE + 2 * E_CHUNK,), jnp.int32),
        pltpu.VMEM((E_CHUNK,), jnp.int32),
        pltpu.VMEM((E_CHUNK,), jnp.int32),
        pltpu.VMEM((1, E_CHUNK), jnp.int32),
        pltpu.VMEM((1, E_CHUNK), jnp.int32),
        pltpu.VMEM((E_CHUNK, HEXT), jnp.float32),
        pltpu.VMEM((E_CHUNK, HEXT), jnp.float32),
        pltpu.SemaphoreType.DMA,
        pltpu.SemaphoreType.DMA,
        pltpu.VMEM((8, HEXT), jnp.float32),
        pltpu.VMEM_SHARED((ACC_ROWS, HEXT), jnp.float32),
    ],
    compiler_params=pltpu.CompilerParams(use_tc_tiling_on_sc=False, needs_layout_passes=False),
)(_sc_agg_body)


def _tc_in_body(xn_ref, cat_ref, wn_ref, bn_ref, wi_ref, bi_ref, out_ref):
    t = jnp.dot(xn_ref[...], wn_ref[...], preferred_element_type=jnp.float32)
    t = jnp.maximum(t + bn_ref[...][None, :], 0.0)
    acc = jnp.dot(t, wi_ref[0:EMBED, :], preferred_element_type=jnp.float32)
    acc += jnp.dot(cat_ref[...], wi_ref[EMBED:, :],
                   preferred_element_type=jnp.float32)
    h = jnp.maximum(acc + bi_ref[...][None, :], 0.0)
    out_ref[:, 0:HIDDEN] = h
    tail = lax.broadcasted_iota(jnp.int32, (ROW_BLK, HEXT - HIDDEN), 1)
    out_ref[:, HIDDEN:] = jnp.where(tail == 0, 1.0, 0.0)


def _tc_in(x_num, cat_concat, W_num, b_num, W_in, b_in):
    return pl.pallas_call(
        _tc_in_body,
        grid=(N_BLKS,),
        in_specs=[
            pl.BlockSpec((ROW_BLK, NUM_NUMERIC), lambda i: (i, 0)),
            pl.BlockSpec((ROW_BLK, NCAT * EMBED), lambda i: (i, 0)),
            pl.BlockSpec((NUM_NUMERIC, EMBED), lambda i: (0, 0)),
            pl.BlockSpec((EMBED,), lambda i: (0,)),
            pl.BlockSpec((EMBED * (1 + NCAT), HIDDEN), lambda i: (0, 0)),
            pl.BlockSpec((HIDDEN,), lambda i: (0,)),
        ],
        out_specs=pl.BlockSpec((ROW_BLK, HEXT), lambda i: (i, 0)),
        out_shape=jax.ShapeDtypeStruct((NPAD, HEXT), jnp.float32),
    )(x_num, cat_concat, W_num, b_num, W_in, b_in)


def _tc_out_body(agg_ref, wg_ref, bg_ref, wo_ref, bo_ref, out_ref):
    a = agg_ref[:, 0:HIDDEN]
    deg = jnp.maximum(agg_ref[:, DEG_COL:DEG_COL + 1], 1.0)
    t = jnp.dot(a, wg_ref[...], preferred_element_type=jnp.float32)
    h = jnp.maximum(t / deg + bg_ref[...][None, :], 0.0)
    o = jnp.sum(h * wo_ref[...][None, :], axis=1) + bo_ref[0]
    out_ref[...] = o


def _tc_out(agg, W_gcn, b_gcn, w_out_vec, b_out):
    return pl.pallas_call(
        _tc_out_body,
        grid=(N_BLKS,),
        in_specs=[
            pl.BlockSpec((ROW_BLK, HEXT), lambda i: (i, 0)),
            pl.BlockSpec((HIDDEN, HIDDEN), lambda i: (0, 0)),
            pl.BlockSpec((HIDDEN,), lambda i: (0,)),
            pl.BlockSpec((HIDDEN,), lambda i: (0,)),
            pl.BlockSpec((128,), lambda i: (0,)),
        ],
        out_specs=pl.BlockSpec((ROW_BLK,), lambda i: (i,)),
        out_shape=jax.ShapeDtypeStruct((NPAD,), jnp.float32),
    )(agg, W_gcn, b_gcn, w_out_vec, b_out)


def kernel(x_num, x_cat, edge_index, tables, W_num, b_num, W_in, b_in,
           W_gcn, b_gcn, W_out, b_out):
    x_num_p = jnp.zeros((NPAD, NUM_NUMERIC), jnp.float32).at[:N].set(x_num)
    x_cat_p = jnp.zeros((NPAD, NCAT), jnp.int32).at[:N].set(
        x_cat.astype(jnp.int32))
    xcat_flat = x_cat_p.reshape(NPAD * NCAT)
    tab_flat = tables.reshape(NCAT * CARD, EMBED)

    src = edge_index[0].astype(jnp.int32)
    dst = edge_index[1].astype(jnp.int32)
    src_p = jnp.zeros((E_PAD,), jnp.int32).at[:E].set(src)
    dst_p = jnp.full((E_PAD,), -8, jnp.int32).at[:E].set(dst)
    edges = jnp.stack([src_p, dst_p])

    cat_flat = _sc_gather(xcat_flat, tab_flat)
    cat_concat = cat_flat.reshape(NPAD, NCAT * EMBED)

    hext = _tc_in(x_num_p, cat_concat, W_num, b_num, W_in, b_in)

    agg = _sc_agg(edges, hext)

    out = _tc_out(agg, W_gcn, b_gcn, W_out[:, 0],
                  jnp.broadcast_to(b_out, (128,)))
    return out[:N]


# speedup vs baseline: 3.9653x; 1.0156x over previous
import functools

import jax
import jax.numpy as jnp
from jax import lax
from jax.experimental import pallas as pl
from jax.experimental.pallas import tpu as pltpu
from jax.experimental.pallas import tpu_sc as plsc

N = 10000
E = 160000
NUM_NUMERIC = 128
EMBED = 128
HIDDEN = 256
NCAT = 8
CARD = 1001

NPAD = 10240
HEXT = HIDDEN + 16
DEG_COL = HIDDEN

NC = 2
NS = 16
LANES = 16

NODES_PER_TILE = NPAD // (NC * NS)
G_CHUNK = 128
CAT_ROWS_PER_TILE = NODES_PER_TILE * NCAT
CAT_CHUNKS = CAT_ROWS_PER_TILE // G_CHUNK

HALF = NPAD // NC
ACC_ROWS = HALF + 8
TRASH = HALF

E_CHUNK = 48
IDX_CHUNK = 1264
E_PER_TILE = -(-E // (NS * IDX_CHUNK)) * IDX_CHUNK
E_PAD = E_PER_TILE * NS
PACK_SHIFT = 14
PACK = 1 << PACK_SHIFT

ROW_BLK = 512
N_BLKS = NPAD // ROW_BLK


def _sc_gather_body(xcatf, tabf, out, idxraw, idxg0, idxg1, row0, row1,
                    sem0, sem1):
    c = lax.axis_index("c")
    s = lax.axis_index("s")
    wid = s * NC + c
    base = wid * CAT_ROWS_PER_TILE
    col_off = (lax.broadcasted_iota(jnp.int32, (LANES,), 0) % NCAT) * CARD

    bufs = [(idxg0, row0, sem0), (idxg1, row1, sem1)]
    descs = [None, None]
    for j in range(CAT_CHUNKS + 1):
        if j < CAT_CHUNKS:
            idxg, rowb, sem = bufs[j % 2]
            off = base + j * G_CHUNK
            pltpu.sync_copy(xcatf.at[pl.ds(off, G_CHUNK)], idxraw)
            for i in range(G_CHUNK // LANES):
                v = idxraw[pl.ds(i * LANES, LANES)]
                v = jnp.minimum(jnp.maximum(v, 0), CARD - 1) + col_off
                idxg[pl.ds(i * LANES, LANES)] = v
            descs[j % 2] = pltpu.async_copy(tabf.at[idxg], rowb, sem)
        if j >= 1:
            k = j - 1
            idxg, rowb, sem = bufs[k % 2]
            descs[k % 2].wait()
            pltpu.sync_copy(rowb, out.at[pl.ds(base + k * G_CHUNK, G_CHUNK)])


_sc_gather = functools.partial(
    pl.kernel,
    out_type=jax.ShapeDtypeStruct((NPAD * NCAT, EMBED), jnp.float32),
    mesh=plsc.VectorSubcoreMesh(core_axis_name="c", subcore_axis_name="s"),
    scratch_types=[
        pltpu.VMEM((G_CHUNK,), jnp.int32),
        pltpu.VMEM((G_CHUNK,), jnp.int32),
        pltpu.VMEM((G_CHUNK,), jnp.int32),
        pltpu.VMEM((G_CHUNK, EMBED), jnp.float32),
        pltpu.VMEM((G_CHUNK, EMBED), jnp.float32),
        pltpu.SemaphoreType.DMA,
        pltpu.SemaphoreType.DMA,
    ],
)(_sc_gather_body)


def _sc_agg_body(edges, hext, agg, srcc, dstc, packed, sidx0, sidx1,
                 dloc0, dloc1, pay0, pay1, gsem0, gsem1, ssem0, ssem1,
                 zbuf, acc):
    c = lax.axis_index("c")
    s = lax.axis_index("s")
    cbase = c * HALF
    ebase = s * E_PER_TILE

    zv = jnp.zeros((LANES,), jnp.float32)
    for r in range(8):
        for i in range(HEXT // LANES):
            zbuf[r, pl.ds(i * LANES, LANES)] = zv
    for t in range(NODES_PER_TILE // 8):
        pltpu.sync_copy(zbuf, acc.at[pl.ds(s * NODES_PER_TILE + t * 8, 8)])

    @pl.when(s == NS - 1)
    def _():
        pltpu.sync_copy(zbuf, acc.at[pl.ds(HALF, ACC_ROWS - HALF)])

    def cblock(q, cursor):
        eoff = pl.multiple_of(ebase + q * IDX_CHUNK, LANES)
        pltpu.sync_copy(edges.at[0, pl.ds(eoff, IDX_CHUNK)], srcc)
        pltpu.sync_copy(edges.at[1, pl.ds(eoff, IDX_CHUNK)], dstc)

        def step(i, cur):
            srcv = srcc[pl.ds(i * LANES, LANES)]
            dstv = dstc[pl.ds(i * LANES, LANES)]
            lv = dstv - cbase
            ok = (lv >= 0) & (lv < HALF)
            pk = srcv * PACK + jnp.where(ok, lv, 0)
            oki = ok.astype(jnp.int32)
            pos = plsc.cumsum(oki)
            plsc.store_scatter(packed, [cur + pos - oki], pk, mask=ok)
            return cur + jnp.max(pos)

        return lax.fori_loop(0, IDX_CHUNK // LANES, step, cursor)

    cursor = lax.fori_loop(0, E_PER_TILE // IDX_CHUNK, cblock, 0)

    trash_pk = TRASH + (lax.broadcasted_iota(jnp.int32, (LANES,), 0) & 7)
    for k in range(2 * E_CHUNK // LANES):
        packed[pl.ds(cursor + k * LANES, LANES)] = trash_pk
    npairs = (cursor + 2 * E_CHUNK - 1) // (2 * E_CHUNK)

    plsc.subcore_barrier()

    bufs = [(sidx0, dloc0, pay0, gsem0, ssem0),
            (sidx1, dloc1, pay1, gsem1, ssem1)]

    def pairbody(p, carry):
        descs = [None, None]
        for b in range(2):
            sidx, dloc, pay, gsem, ssem = bufs[b]

            @pl.when(p > 0)
            def _():
                pltpu.make_async_copy(
                    hext.at[pl.ds(0, E_CHUNK)], pay, ssem).wait()

            off = pl.multiple_of(p * 2 * E_CHUNK + b * E_CHUNK, E_CHUNK)
            for i in range(E_CHUNK // LANES):
                pk = packed[pl.ds(off + i * LANES, LANES)]
                sidx[pl.ds(i * LANES, LANES)] = lax.shift_right_logical(
                    pk, PACK_SHIFT)
                dloc[0, pl.ds(i * LANES, LANES)] = pk & (PACK - 1)
            descs[b] = pltpu.async_copy(hext.at[sidx], pay, gsem)
        for b in range(2):
            sidx, dloc, pay, gsem, ssem = bufs[b]
            descs[b].wait()
            pltpu.async_copy(pay, acc.at[dloc.at[0]], ssem, add=True)
        return carry

    lax.fori_loop(0, npairs, pairbody, 0)

    @pl.when(npairs > 0)
    def _():
        for b in range(2):
            _, _, pay, _, ssem = bufs[b]
            pltpu.make_async_copy(
                hext.at[pl.ds(0, E_CHUNK)], pay, ssem).wait()

    plsc.subcore_barrier()
    pltpu.sync_copy(
        acc.at[pl.ds(s * NODES_PER_TILE, NODES_PER_TILE)],
        agg.at[pl.ds(cbase + s * NODES_PER_TILE, NODES_PER_TILE)])


_sc_agg = functools.partial(
    pl.kernel,
    out_type=jax.ShapeDtypeStruct((NPAD, HEXT), jnp.float32),
    mesh=plsc.VectorSubcoreMesh(core_axis_name="c", subcore_axis_name="s"),
    scratch_types=[
        pltpu.VMEM((IDX_CHUNK,), jnp.int32),
        pltpu.VMEM((IDX_CHUNK,), jnp.int32),
        pltpu.VMEM((E_PER_TILE + 2 * E_CHUNK,), jnp.int32),
        pltpu.VMEM((E_CHUNK,), jnp.int32),
        pltpu.VMEM((E_CHUNK,), jnp.int32),
        pltpu.VMEM((1, E_CHUNK), jnp.int32),
        pltpu.VMEM((1, E_CHUNK), jnp.int32),
        pltpu.VMEM((E_CHUNK, HEXT), jnp.float32),
        pltpu.VMEM((E_CHUNK, HEXT), jnp.float32),
        pltpu.SemaphoreType.DMA,
        pltpu.SemaphoreType.DMA,
        pltpu.SemaphoreType.DMA,
        pltpu.SemaphoreType.DMA,
        pltpu.VMEM((8, HEXT), jnp.float32),
        pltpu.VMEM_SHARED((ACC_ROWS, HEXT), jnp.float32),
    ],
    compiler_params=pltpu.CompilerParams(use_tc_tiling_on_sc=False, needs_layout_passes=False),
)(_sc_agg_body)


def _tc_in_body(xn_ref, cat_ref, wn_ref, bn_ref, wi_ref, bi_ref, out_ref):
    t = jnp.dot(xn_ref[...], wn_ref[...], preferred_element_type=jnp.float32)
    t = jnp.maximum(t + bn_ref[...][None, :], 0.0)
    acc = jnp.dot(t, wi_ref[0:EMBED, :], preferred_element_type=jnp.float32)
    acc += jnp.dot(cat_ref[...], wi_ref[EMBED:, :],
                   preferred_element_type=jnp.float32)
    h = jnp.maximum(acc + bi_ref[...][None, :], 0.0)
    out_ref[:, 0:HIDDEN] = h
    tail = lax.broadcasted_iota(jnp.int32, (ROW_BLK, HEXT - HIDDEN), 1)
    out_ref[:, HIDDEN:] = jnp.where(tail == 0, 1.0, 0.0)


def _tc_in(x_num, cat_concat, W_num, b_num, W_in, b_in):
    return pl.pallas_call(
        _tc_in_body,
        grid=(N_BLKS,),
        in_specs=[
            pl.BlockSpec((ROW_BLK, NUM_NUMERIC), lambda i: (i, 0)),
            pl.BlockSpec((ROW_BLK, NCAT * EMBED), lambda i: (i, 0)),
            pl.BlockSpec((NUM_NUMERIC, EMBED), lambda i: (0, 0)),
            pl.BlockSpec((EMBED,), lambda i: (0,)),
            pl.BlockSpec((EMBED * (1 + NCAT), HIDDEN), lambda i: (0, 0)),
            pl.BlockSpec((HIDDEN,), lambda i: (0,)),
        ],
        out_specs=pl.BlockSpec((ROW_BLK, HEXT), lambda i: (i, 0)),
        out_shape=jax.ShapeDtypeStruct((NPAD, HEXT), jnp.float32),
    )(x_num, cat_concat, W_num, b_num, W_in, b_in)


def _tc_out_body(agg_ref, wg_ref, bg_ref, wo_ref, bo_ref, out_ref):
    deg = jnp.maximum(agg_ref[:, DEG_COL:DEG_COL + 1], 1.0)
    a = agg_ref[:, 0:HIDDEN] / deg
    t = jnp.dot(a, wg_ref[...], preferred_element_type=jnp.float32)
    h = jnp.maximum(t + bg_ref[...][None, :], 0.0)
    o = jnp.dot(h, wo_ref[...], preferred_element_type=jnp.float32)
    out_ref[...] = o[:, 0] + bo_ref[0]


def _tc_out(agg, W_gcn, b_gcn, w_out_vec, b_out):
    return pl.pallas_call(
        _tc_out_body,
        grid=(N_BLKS,),
        in_specs=[
            pl.BlockSpec((ROW_BLK, HEXT), lambda i: (i, 0)),
            pl.BlockSpec((HIDDEN, HIDDEN), lambda i: (0, 0)),
            pl.BlockSpec((HIDDEN,), lambda i: (0,)),
            pl.BlockSpec((HIDDEN, 128), lambda i: (0, 0)),
            pl.BlockSpec((128,), lambda i: (0,)),
        ],
        out_specs=pl.BlockSpec((ROW_BLK,), lambda i: (i,)),
        out_shape=jax.ShapeDtypeStruct((NPAD,), jnp.float32),
    )(agg, W_gcn, b_gcn, w_out_vec, b_out)


def kernel(x_num, x_cat, edge_index, tables, W_num, b_num, W_in, b_in,
           W_gcn, b_gcn, W_out, b_out):
    x_num_p = jnp.zeros((NPAD, NUM_NUMERIC), jnp.float32).at[:N].set(x_num)
    x_cat_p = jnp.zeros((NPAD, NCAT), jnp.int32).at[:N].set(
        x_cat.astype(jnp.int32))
    xcat_flat = x_cat_p.reshape(NPAD * NCAT)
    tab_flat = tables.reshape(NCAT * CARD, EMBED)

    src = edge_index[0].astype(jnp.int32)
    dst = edge_index[1].astype(jnp.int32)
    src_p = jnp.zeros((E_PAD,), jnp.int32).at[:E].set(src)
    dst_p = jnp.full((E_PAD,), -8, jnp.int32).at[:E].set(dst)
    edges = jnp.stack([src_p, dst_p])

    cat_flat = _sc_gather(xcat_flat, tab_flat)
    cat_concat = cat_flat.reshape(NPAD, NCAT * EMBED)

    hext = _tc_in(x_num_p, cat_concat, W_num, b_num, W_in, b_in)

    agg = _sc_agg(edges, hext)

    w_out_pad = jnp.zeros((HIDDEN, 128), jnp.float32).at[:, 0].set(W_out[:, 0])
    out = _tc_out(agg, W_gcn, b_gcn, w_out_pad,
                  jnp.broadcast_to(b_out, (128,)))
    return out[:N]


# speedup vs baseline: 4.1253x; 1.0403x over previous
import functools

import jax
import jax.numpy as jnp
from jax import lax
from jax.experimental import pallas as pl
from jax.experimental.pallas import tpu as pltpu
from jax.experimental.pallas import tpu_sc as plsc

N = 10000
E = 160000
NUM_NUMERIC = 128
EMBED = 128
HIDDEN = 256
NCAT = 8
CARD = 1001

NPAD = 10240
HEXT = HIDDEN + 16
DEG_COL = HIDDEN

NC = 2
NS = 16
LANES = 16

NODES_PER_TILE = NPAD // (NC * NS)
G_CHUNK = 128
CAT_ROWS_PER_TILE = NODES_PER_TILE * NCAT
CAT_CHUNKS = CAT_ROWS_PER_TILE // G_CHUNK

HALF = NPAD // NC
ACC_ROWS = HALF + 8
TRASH = HALF

E_CHUNK = 48
IDX_CHUNK = 2000
E_PER_TILE = E // NS
PACK_SHIFT = 14
PACK = 1 << PACK_SHIFT

ROW_BLK = 512
N_BLKS = NPAD // ROW_BLK


def _sc_gather_body(xcatf, tabf, out, idxraw, idxg0, idxg1, row0, row1,
                    sem0, sem1):
    c = lax.axis_index("c")
    s = lax.axis_index("s")
    wid = s * NC + c
    base = wid * CAT_ROWS_PER_TILE
    col_off = (lax.broadcasted_iota(jnp.int32, (LANES,), 0) % NCAT) * CARD

    bufs = [(idxg0, row0, sem0), (idxg1, row1, sem1)]
    descs = [None, None]
    for j in range(CAT_CHUNKS + 1):
        if j < CAT_CHUNKS:
            idxg, rowb, sem = bufs[j % 2]
            off = base + j * G_CHUNK
            pltpu.sync_copy(xcatf.at[pl.ds(off, G_CHUNK)], idxraw)
            for i in range(G_CHUNK // LANES):
                v = idxraw[pl.ds(i * LANES, LANES)]
                v = jnp.minimum(jnp.maximum(v, 0), CARD - 1) + col_off
                idxg[pl.ds(i * LANES, LANES)] = v
            descs[j % 2] = pltpu.async_copy(tabf.at[idxg], rowb, sem)
        if j >= 1:
            k = j - 1
            idxg, rowb, sem = bufs[k % 2]
            descs[k % 2].wait()
            pltpu.sync_copy(rowb, out.at[pl.ds(base + k * G_CHUNK, G_CHUNK)])


_sc_gather = functools.partial(
    pl.kernel,
    out_type=jax.ShapeDtypeStruct((NPAD * NCAT, EMBED), jnp.float32),
    mesh=plsc.VectorSubcoreMesh(core_axis_name="c", subcore_axis_name="s"),
    scratch_types=[
        pltpu.VMEM((G_CHUNK,), jnp.int32),
        pltpu.VMEM((G_CHUNK,), jnp.int32),
        pltpu.VMEM((G_CHUNK,), jnp.int32),
        pltpu.VMEM((G_CHUNK, EMBED), jnp.float32),
        pltpu.VMEM((G_CHUNK, EMBED), jnp.float32),
        pltpu.SemaphoreType.DMA,
        pltpu.SemaphoreType.DMA,
    ],
)(_sc_gather_body)


def _sc_agg_body(edges, hext, agg, srcc, dstc, packed, sidx0, sidx1,
                 dloc0, dloc1, pay0, pay1, gsem0, gsem1, ssem0, ssem1,
                 zbuf, acc):
    c = lax.axis_index("c")
    s = lax.axis_index("s")
    cbase = c * HALF
    ebase = s * E_PER_TILE

    zv = jnp.zeros((LANES,), jnp.float32)
    for r in range(8):
        for i in range(HEXT // LANES):
            zbuf[r, pl.ds(i * LANES, LANES)] = zv
    for t in range(NODES_PER_TILE // 8):
        pltpu.sync_copy(zbuf, acc.at[pl.ds(s * NODES_PER_TILE + t * 8, 8)])

    @pl.when(s == NS - 1)
    def _():
        pltpu.sync_copy(zbuf, acc.at[pl.ds(HALF, ACC_ROWS - HALF)])

    def cblock(q, cursor):
        eoff = pl.multiple_of(ebase + q * IDX_CHUNK, LANES)
        pltpu.sync_copy(edges.at[0, pl.ds(eoff, IDX_CHUNK)], srcc)
        pltpu.sync_copy(edges.at[1, pl.ds(eoff, IDX_CHUNK)], dstc)

        def step(i, cur):
            srcv = srcc[pl.ds(i * LANES, LANES)]
            dstv = dstc[pl.ds(i * LANES, LANES)]
            lv = dstv - cbase
            ok = (lv >= 0) & (lv < HALF)
            pk = srcv * PACK + jnp.where(ok, lv, 0)
            oki = ok.astype(jnp.int32)
            pos = plsc.cumsum(oki)
            plsc.store_scatter(packed, [cur + pos - oki], pk, mask=ok)
            return cur + jnp.max(pos)

        return lax.fori_loop(0, IDX_CHUNK // LANES, step, cursor)

    cursor = lax.fori_loop(0, E_PER_TILE // IDX_CHUNK, cblock, 0)

    trash_pk = TRASH + (lax.broadcasted_iota(jnp.int32, (LANES,), 0) & 7)
    for k in range(2 * E_CHUNK // LANES):
        packed[pl.ds(cursor + k * LANES, LANES)] = trash_pk
    npairs = (cursor + 2 * E_CHUNK - 1) // (2 * E_CHUNK)

    plsc.subcore_barrier()

    bufs = [(sidx0, dloc0, pay0, gsem0, ssem0),
            (sidx1, dloc1, pay1, gsem1, ssem1)]

    def pairbody(p, carry):
        descs = [None, None]
        for b in range(2):
            sidx, dloc, pay, gsem, ssem = bufs[b]

            @pl.when(p > 0)
            def _():
                pltpu.make_async_copy(
                    hext.at[pl.ds(0, E_CHUNK)], pay, ssem).wait()

            off = pl.multiple_of(p * 2 * E_CHUNK + b * E_CHUNK, E_CHUNK)
            for i in range(E_CHUNK // LANES):
                pk = packed[pl.ds(off + i * LANES, LANES)]
                sidx[pl.ds(i * LANES, LANES)] = lax.shift_right_logical(
                    pk, PACK_SHIFT)
                dloc[0, pl.ds(i * LANES, LANES)] = pk & (PACK - 1)
            descs[b] = pltpu.async_copy(hext.at[sidx], pay, gsem)
        for b in range(2):
            sidx, dloc, pay, gsem, ssem = bufs[b]
            descs[b].wait()
            pltpu.async_copy(pay, acc.at[dloc.at[0]], ssem, add=True)
        return carry

    lax.fori_loop(0, npairs, pairbody, 0)

    @pl.when(npairs > 0)
    def _():
        for b in range(2):
            _, _, pay, _, ssem = bufs[b]
            pltpu.make_async_copy(
                hext.at[pl.ds(0, E_CHUNK)], pay, ssem).wait()

    plsc.subcore_barrier()
    pltpu.sync_copy(
        acc.at[pl.ds(s * NODES_PER_TILE, NODES_PER_TILE)],
        agg.at[pl.ds(cbase + s * NODES_PER_TILE, NODES_PER_TILE)])


_sc_agg = functools.partial(
    pl.kernel,
    out_type=jax.ShapeDtypeStruct((NPAD, HEXT), jnp.float32),
    mesh=plsc.VectorSubcoreMesh(core_axis_name="c", subcore_axis_name="s"),
    scratch_types=[
        pltpu.VMEM((IDX_CHUNK,), jnp.int32),
        pltpu.VMEM((IDX_CHUNK,), jnp.int32),
        pltpu.VMEM((E_PER_TILE + 2 * E_CHUNK,), jnp.int32),
        pltpu.VMEM((E_CHUNK,), jnp.int32),
        pltpu.VMEM((E_CHUNK,), jnp.int32),
        pltpu.VMEM((1, E_CHUNK), jnp.int32),
        pltpu.VMEM((1, E_CHUNK), jnp.int32),
        pltpu.VMEM((E_CHUNK, HEXT), jnp.float32),
        pltpu.VMEM((E_CHUNK, HEXT), jnp.float32),
        pltpu.SemaphoreType.DMA,
        pltpu.SemaphoreType.DMA,
        pltpu.SemaphoreType.DMA,
        pltpu.SemaphoreType.DMA,
        pltpu.VMEM((8, HEXT), jnp.float32),
        pltpu.VMEM_SHARED((ACC_ROWS, HEXT), jnp.float32),
    ],
    compiler_params=pltpu.CompilerParams(use_tc_tiling_on_sc=False, needs_layout_passes=False),
)(_sc_agg_body)


def _tc_in_body(xn_ref, cat_ref, wn_ref, bn_ref, wi_ref, bi_ref, out_ref):
    t = jnp.dot(xn_ref[...], wn_ref[...], preferred_element_type=jnp.float32)
    t = jnp.maximum(t + bn_ref[...][None, :], 0.0)
    acc = jnp.dot(t, wi_ref[0:EMBED, :], preferred_element_type=jnp.float32)
    for cc in range(NCAT):
        acc += jnp.dot(cat_ref[:, cc, :],
                       wi_ref[EMBED * (1 + cc):EMBED * (2 + cc), :],
                       preferred_element_type=jnp.float32)
    h = jnp.maximum(acc + bi_ref[...][None, :], 0.0)
    out_ref[:, 0:HIDDEN] = h
    tail = lax.broadcasted_iota(jnp.int32, (ROW_BLK, HEXT - HIDDEN), 1)
    out_ref[:, HIDDEN:] = jnp.where(tail == 0, 1.0, 0.0)


def _tc_in(x_num, cat_concat, W_num, b_num, W_in, b_in):
    return pl.pallas_call(
        _tc_in_body,
        grid=(N_BLKS,),
        in_specs=[
            pl.BlockSpec((ROW_BLK, NUM_NUMERIC), lambda i: (i, 0)),
            pl.BlockSpec((ROW_BLK, NCAT, EMBED), lambda i: (i, 0, 0)),
            pl.BlockSpec((NUM_NUMERIC, EMBED), lambda i: (0, 0)),
            pl.BlockSpec((EMBED,), lambda i: (0,)),
            pl.BlockSpec((EMBED * (1 + NCAT), HIDDEN), lambda i: (0, 0)),
            pl.BlockSpec((HIDDEN,), lambda i: (0,)),
        ],
        out_specs=pl.BlockSpec((ROW_BLK, HEXT), lambda i: (i, 0)),
        out_shape=jax.ShapeDtypeStruct((NPAD, HEXT), jnp.float32),
    )(x_num, cat_concat, W_num, b_num, W_in, b_in)


def _tc_out_body(agg_ref, wg_ref, bg_ref, wo_ref, bo_ref, out_ref):
    deg = jnp.maximum(agg_ref[:, DEG_COL:DEG_COL + 1], 1.0)
    a = agg_ref[:, 0:HIDDEN] / deg
    t = jnp.dot(a, wg_ref[...], preferred_element_type=jnp.float32)
    h = jnp.maximum(t + bg_ref[...][None, :], 0.0)
    o = jnp.dot(h, wo_ref[...], preferred_element_type=jnp.float32)
    out_ref[...] = o[:, 0] + bo_ref[0]


def _tc_out(agg, W_gcn, b_gcn, w_out_vec, b_out):
    return pl.pallas_call(
        _tc_out_body,
        grid=(N_BLKS,),
        in_specs=[
            pl.BlockSpec((ROW_BLK, HEXT), lambda i: (i, 0)),
            pl.BlockSpec((HIDDEN, HIDDEN), lambda i: (0, 0)),
            pl.BlockSpec((HIDDEN,), lambda i: (0,)),
            pl.BlockSpec((HIDDEN, 128), lambda i: (0, 0)),
            pl.BlockSpec((128,), lambda i: (0,)),
        ],
        out_specs=pl.BlockSpec((ROW_BLK,), lambda i: (i,)),
        out_shape=jax.ShapeDtypeStruct((NPAD,), jnp.float32),
    )(agg, W_gcn, b_gcn, w_out_vec, b_out)


def kernel(x_num, x_cat, edge_index, tables, W_num, b_num, W_in, b_in,
           W_gcn, b_gcn, W_out, b_out):
    x_num_p = jnp.zeros((NPAD, NUM_NUMERIC), jnp.float32).at[:N].set(x_num)
    x_cat_p = jnp.zeros((NPAD, NCAT), jnp.int32).at[:N].set(
        x_cat.astype(jnp.int32))
    xcat_flat = x_cat_p.reshape(NPAD * NCAT)
    tab_flat = tables.reshape(NCAT * CARD, EMBED)

    edges = edge_index.astype(jnp.int32)

    cat_flat = _sc_gather(xcat_flat, tab_flat)
    cat3 = cat_flat.reshape(NPAD, NCAT, EMBED)

    hext = _tc_in(x_num_p, cat3, W_num, b_num, W_in, b_in)

    agg = _sc_agg(edges, hext)

    w_out_pad = jnp.zeros((HIDDEN, 128), jnp.float32).at[:, 0].set(W_out[:, 0])
    out = _tc_out(agg, W_gcn, b_gcn, w_out_pad,
                  jnp.broadcast_to(b_out, (128,)))
    return out[:N]
